# baseline (speedup 1.0000x reference)
import functools
import math

import jax
import jax.numpy as jnp
from jax import lax
from jax.experimental import pallas as pl
from jax.experimental.pallas import tpu as pltpu
from jax.experimental.pallas import tpu_sc as plsc

S = 200
B = 1024
D = 64
N = S * B
MAX_LEN = 1000

NC = 2
NS = 16
NW = NC * NS
CHUNK = 256
NCHUNK = N // CHUNK
IDXW = 128
QPC = CHUNK // IDXW
T = NCHUNK // NW
RSTEP = 2
NJ = D // 16


def _make_pe():
    position = jnp.arange(MAX_LEN, dtype=jnp.float32)[:, None]
    div_term = jnp.exp(
        jnp.arange(0, D, 2, dtype=jnp.float32) * -(math.log(10000.0) / D))
    ang = position * div_term
    pe = jnp.zeros((MAX_LEN, D), dtype=jnp.float32)
    pe = pe.at[:, 0::2].set(jnp.sin(ang))
    pe = pe.at[:, 1::2].set(jnp.cos(ang))
    return pe[:S]


_mesh = plsc.VectorSubcoreMesh(core_axis_name="c", subcore_axis_name="s")


@functools.partial(
    pl.kernel,
    mesh=_mesh,
    compiler_params=pltpu.CompilerParams(use_tc_tiling_on_sc=False),
    out_type=jax.ShapeDtypeStruct((N // 2, 2 * D), jnp.float32),
    scratch_types=[
        pltpu.VMEM((T * QPC, IDXW), jnp.int32),
        pltpu.VMEM((2, CHUNK, D), jnp.float32),
        pltpu.VMEM((2, CHUNK // 2, 2 * D), jnp.float32),
        pltpu.VMEM((S, D), jnp.float32),
        pltpu.SemaphoreType.DMA,
        pltpu.SemaphoreType.DMA,
        pltpu.SemaphoreType.DMA,
    ],
)
def _emb(idx_hbm, table_hbm, pe_hbm, out_hbm, idx_v, rows_v, pack_v, pe_v,
         sem_i, sem_g, sem_o):
    wid = lax.axis_index("s") * NC + lax.axis_index("c")
    icp = pltpu.async_copy(idx_hbm.at[wid], idx_v, sem_i)
    pltpu.sync_copy(pe_hbm, pe_v)
    icp.wait()

    @plsc.parallel_loop(0, T * QPC, step=1, unroll=2)
    def _(r):
        for jj in range(IDXW // 16):
            v = idx_v[r, pl.ds(16 * jj, 16)]
            idx_v[r, pl.ds(16 * jj, 16)] = (
                (v & (-BT)) + ((v & (BT // 2 - 1)) << 1)
                + ((v >> HBIT) & 1))

    def chunk_id(t):
        return wid + NW * t

    def fire_gathers(t):
        p = t % 2
        return [
            pltpu.async_copy(
                table_hbm.at[idx_v.at[t * QPC + q]],
                rows_v.at[p, pl.ds(q * IDXW, IDXW)], sem_g)
            for q in range(QPC)
        ]

    def add_pe(t):
        p = t % 2
        s_c = chunk_id(t) * CHUNK // B
        pe_row = [pe_v[s_c, pl.ds(16 * j, 16)] for j in range(NJ)]

        @plsc.parallel_loop(0, CHUNK // 2, step=RSTEP, unroll=2)
        def _(r):
            for rr in range(RSTEP):
                for half in range(2):
                    for j in range(NJ):
                        pack_v[p, r + rr, pl.ds(64 * half + 16 * j, 16)] = (
                            rows_v[p, 2 * (r + rr) + half, pl.ds(16 * j, 16)]
                            + pe_row[j])

    def fire_out(t):
        return pltpu.async_copy(
            pack_v.at[t % 2],
            out_hbm.at[pl.ds(chunk_id(t) * (CHUNK // 2), CHUNK // 2)], sem_o)

    g_cps = {0: fire_gathers(0)}
    o_cps = {}

    for t in range(T):
        for cp in g_cps.pop(t):
            cp.wait()
        if t - 1 in o_cps:
            o_cps.pop(t - 1).wait()
        if t + 1 < T:
            g_cps[t + 1] = fire_gathers(t + 1)
        add_pe(t)
        o_cps[t] = fire_out(t)

    o_cps.pop(T - 1).wait()


VOCAB = 1000000
BT = 16384
NBLK = -(-VOCAB // BT)
HBIT = (BT // 2).bit_length() - 1


def _repack_block(wt_ref, o_ref):
    x = wt_ref[...]
    o_ref[:, :D] = x[:, :BT // 2].T
    o_ref[:, D:] = x[:, BT // 2:].T


def _repack(wt):
    return pl.pallas_call(
        _repack_block,
        grid=(NBLK,),
        in_specs=[pl.BlockSpec((D, BT), lambda j: (0, j))],
        out_specs=pl.BlockSpec((BT // 2, 2 * D), lambda j: (j, 0)),
        out_shape=jax.ShapeDtypeStruct((NBLK * BT // 2, 2 * D), jnp.float32),
    )(wt)


def kernel(sequence, W):
    idx = (sequence.reshape(T, NW, QPC, IDXW)
           .transpose(1, 0, 2, 3)
           .reshape(NW, T * QPC, IDXW))
    w_lin = _repack(W.T).reshape(NBLK * BT, D)
    pe = _make_pe()
    out = _emb(idx, w_lin, pe)
    return out.reshape(S, B, D)


# speedup vs baseline: 1.0405x; 1.0405x over previous
import functools
import math

import jax
import jax.numpy as jnp
from jax import lax
from jax.experimental import pallas as pl
from jax.experimental.pallas import tpu as pltpu
from jax.experimental.pallas import tpu_sc as plsc

S = 200
B = 1024
D = 64
N = S * B
MAX_LEN = 1000

NC = 2
NS = 16
NW = NC * NS
CHUNK = 256
NCHUNK = N // CHUNK
IDXW = 128
QPC = CHUNK // IDXW
T = NCHUNK // NW
RSTEP = 2
NJ = D // 16


def _make_pe():
    position = jnp.arange(MAX_LEN, dtype=jnp.float32)[:, None]
    div_term = jnp.exp(
        jnp.arange(0, D, 2, dtype=jnp.float32) * -(math.log(10000.0) / D))
    ang = position * div_term
    pe = jnp.zeros((MAX_LEN, D), dtype=jnp.float32)
    pe = pe.at[:, 0::2].set(jnp.sin(ang))
    pe = pe.at[:, 1::2].set(jnp.cos(ang))
    return pe[:S]


_mesh = plsc.VectorSubcoreMesh(core_axis_name="c", subcore_axis_name="s")


@functools.partial(
    pl.kernel,
    mesh=_mesh,
    compiler_params=pltpu.CompilerParams(use_tc_tiling_on_sc=False),
    out_type=jax.ShapeDtypeStruct((N // 2, 2 * D), jnp.float32),
    scratch_types=[
        pltpu.VMEM((T * QPC, IDXW), jnp.int32),
        pltpu.VMEM((2, CHUNK, D), jnp.float32),
        pltpu.VMEM((2, CHUNK // 2, 2 * D), jnp.float32),
        pltpu.VMEM((S, D), jnp.float32),
        pltpu.SemaphoreType.DMA,
        pltpu.SemaphoreType.DMA,
        pltpu.SemaphoreType.DMA,
    ],
)
def _emb(idx_hbm, table_hbm, pe_hbm, out_hbm, idx_v, rows_v, pack_v, pe_v,
         sem_i, sem_g, sem_o):
    wid = lax.axis_index("s") * NC + lax.axis_index("c")
    icp = pltpu.async_copy(idx_hbm.at[wid], idx_v, sem_i)
    pltpu.sync_copy(pe_hbm, pe_v)
    icp.wait()

    @plsc.parallel_loop(0, T * QPC, step=1, unroll=2)
    def _(r):
        for jj in range(IDXW // 16):
            v = idx_v[r, pl.ds(16 * jj, 16)]
            idx_v[r, pl.ds(16 * jj, 16)] = (
                (v & (-BT)) + ((v & (BT // 2 - 1)) << 1)
                + ((v >> HBIT) & 1))

    def chunk_id(t):
        return wid + NW * t

    def fire_gathers(t):
        p = t % 2
        return [
            pltpu.async_copy(
                table_hbm.at[idx_v.at[t * QPC + q]],
                rows_v.at[p, pl.ds(q * IDXW, IDXW)], sem_g)
            for q in range(QPC)
        ]

    def add_pe(t):
        p = t % 2
        s_c = chunk_id(t) * CHUNK // B
        pe_row = [pe_v[s_c, pl.ds(16 * j, 16)] for j in range(NJ)]

        @plsc.parallel_loop(0, CHUNK // 2, step=RSTEP, unroll=2)
        def _(r):
            for rr in range(RSTEP):
                for half in range(2):
                    for j in range(NJ):
                        pack_v[p, r + rr, pl.ds(64 * half + 16 * j, 16)] = (
                            rows_v[p, 2 * (r + rr) + half, pl.ds(16 * j, 16)]
                            + pe_row[j])

    def fire_out(t):
        return pltpu.async_copy(
            pack_v.at[t % 2],
            out_hbm.at[pl.ds(chunk_id(t) * (CHUNK // 2), CHUNK // 2)], sem_o)

    g_cps = {0: fire_gathers(0)}
    o_cps = {}

    for t in range(T):
        for cp in g_cps.pop(t):
            cp.wait()
        if t - 1 in o_cps:
            o_cps.pop(t - 1).wait()
        if t + 1 < T:
            g_cps[t + 1] = fire_gathers(t + 1)
        add_pe(t)
        o_cps[t] = fire_out(t)

    o_cps.pop(T - 1).wait()


VOCAB = 1000000
BT = 32768
NBLK = -(-VOCAB // BT)
HBIT = (BT // 2).bit_length() - 1


def _repack_block(wt_ref, o_ref):
    x = wt_ref[...]
    o_ref[:, :D] = x[:, :BT // 2].T
    o_ref[:, D:] = x[:, BT // 2:].T


def _repack(wt):
    return pl.pallas_call(
        _repack_block,
        grid=(NBLK,),
        in_specs=[pl.BlockSpec((D, BT), lambda j: (0, j))],
        out_specs=pl.BlockSpec((BT // 2, 2 * D), lambda j: (j, 0)),
        out_shape=jax.ShapeDtypeStruct((NBLK * BT // 2, 2 * D), jnp.float32),
    )(wt)


def kernel(sequence, W):
    idx = (sequence.reshape(T, NW, QPC, IDXW)
           .transpose(1, 0, 2, 3)
           .reshape(NW, T * QPC, IDXW))
    w_lin = _repack(W.T).reshape(NBLK * BT, D)
    pe = _make_pe()
    out = _emb(idx, w_lin, pe)
    return out.reshape(S, B, D)
